# baseline (speedup 1.0000x reference)
import jax
import jax.numpy as jnp
from jax.experimental import pallas as pl
from jax.experimental.pallas import tpu as pltpu

LATENT_DIM = 1024
CODEBOOK_SIZE = 1024
BETA = 0.25
PIX = 1024
NUM_BLOCKS = 8
TOTAL_ROWS = NUM_BLOCKS * PIX


def _vq_kernel(x_ref, w_ref, qst_ref, enc_ref, idx_ref, loss_ref, perp_ref,
               acc_ref, cnt_ref, colsq_ref):
    i = pl.program_id(0)
    xb = x_ref[...]
    w = w_ref[...]

    @pl.when(i == 0)
    def _init():
        colsq_ref[...] = jnp.sum(w * w, axis=0, keepdims=True).T
        acc_ref[...] = jnp.zeros_like(acc_ref)
        cnt_ref[...] = jnp.zeros_like(cnt_ref)

    xsq = jnp.sum(xb * xb, axis=0, keepdims=True)
    st = jax.lax.dot_general(w, xb, (((1,), (0,)), ((), ())))
    dist = (xsq + colsq_ref[...]) - 2.0 * st

    dmin = jnp.min(dist, axis=0, keepdims=True)
    iota0 = jax.lax.broadcasted_iota(jnp.int32, dist.shape, 0)
    idx_row = jnp.min(jnp.where(dist == dmin, iota0, CODEBOOK_SIZE),
                      axis=0, keepdims=True)

    idx_col = idx_row.T
    iota1 = jax.lax.broadcasted_iota(jnp.int32, (PIX, CODEBOOK_SIZE), 1)
    enc = (iota1 == idx_col).astype(jnp.float32)
    enc_ref[...] = enc
    idx_ref[...] = idx_col

    qst_ref[...] = jax.lax.dot_general(enc, w, (((1,), (0,)), ((), ())))

    acc_ref[...] += jnp.sum(dmin, keepdims=True)
    cnt_ref[...] += jnp.sum(enc, axis=0, keepdims=True)

    @pl.when(i == NUM_BLOCKS - 1)
    def _finalize():
        rowsq = jnp.sum(w * w, axis=1, keepdims=True)
        cnt = cnt_ref[...]
        corr = (jax.lax.dot_general(cnt, rowsq, (((1,), (0,)), ((), ())))
                - jnp.sum(cnt * colsq_ref[...].T, keepdims=True))
        m = (acc_ref[...] + corr) / jnp.float32(TOTAL_ROWS * LATENT_DIM)
        loss_ref[...] = m + jnp.float32(BETA) * m
        avg = cnt / jnp.float32(TOTAL_ROWS)
        ent = jnp.sum(avg * jnp.log(avg + 1e-10), keepdims=True)
        perp_ref[...] = jnp.exp(-ent)


@jax.jit
def kernel(x, W):
    xr = x.reshape(NUM_BLOCKS * LATENT_DIM, PIX)

    out_shapes = (
        jax.ShapeDtypeStruct((TOTAL_ROWS, LATENT_DIM), jnp.float32),
        jax.ShapeDtypeStruct((TOTAL_ROWS, CODEBOOK_SIZE), jnp.float32),
        jax.ShapeDtypeStruct((TOTAL_ROWS, 1), jnp.int32),
        jax.ShapeDtypeStruct((1, 1), jnp.float32),
        jax.ShapeDtypeStruct((1, 1), jnp.float32),
    )
    qst, enc, idx, loss, perp = pl.pallas_call(
        _vq_kernel,
        grid=(NUM_BLOCKS,),
        in_specs=[
            pl.BlockSpec((LATENT_DIM, PIX), lambda i: (i, 0)),
            pl.BlockSpec((CODEBOOK_SIZE, LATENT_DIM), lambda i: (0, 0)),
        ],
        out_specs=(
            pl.BlockSpec((PIX, LATENT_DIM), lambda i: (i, 0)),
            pl.BlockSpec((PIX, CODEBOOK_SIZE), lambda i: (i, 0)),
            pl.BlockSpec((PIX, 1), lambda i: (i, 0)),
            pl.BlockSpec((1, 1), lambda i: (0, 0)),
            pl.BlockSpec((1, 1), lambda i: (0, 0)),
        ),
        scratch_shapes=[
            pltpu.VMEM((1, 1), jnp.float32),
            pltpu.VMEM((1, CODEBOOK_SIZE), jnp.float32),
            pltpu.VMEM((CODEBOOK_SIZE, 1), jnp.float32),
        ],
        out_shape=out_shapes,
    )(xr, W)

    q_out = jnp.transpose(qst.reshape(NUM_BLOCKS, 32, 32, LATENT_DIM),
                          (0, 3, 1, 2))
    return (loss[0, 0], q_out, perp[0, 0], enc, idx)


# speedup vs baseline: 2.4996x; 2.4996x over previous
import jax
import jax.numpy as jnp
from jax.experimental import pallas as pl
from jax.experimental.pallas import tpu as pltpu

LATENT_DIM = 1024
CODEBOOK_SIZE = 1024
BETA = 0.25
BLOCK_ROWS = 1024
TOTAL_ROWS = 8192
NUM_BLOCKS = TOTAL_ROWS // BLOCK_ROWS


def _vq_kernel(x_ref, w_ref, qst_ref, enc_ref, idx_ref, loss_ref, perp_ref,
               acc_ref, cnt_ref, colsq_ref):
    i = pl.program_id(0)
    xb = x_ref[...]
    w = w_ref[...]

    @pl.when(i == 0)
    def _init():
        colsq_ref[...] = jnp.sum(w * w, axis=0, keepdims=True)
        acc_ref[...] = jnp.zeros_like(acc_ref)
        cnt_ref[...] = jnp.zeros_like(cnt_ref)

    xsq = jnp.sum(xb * xb, axis=-1, keepdims=True)
    s = jax.lax.dot_general(xb, w, (((1,), (1,)), ((), ())))
    distances = (xsq + colsq_ref[...]) - 2.0 * s

    dmin = jnp.min(distances, axis=1, keepdims=True)
    idx2d = jnp.argmin(distances, axis=1)[:, None].astype(jnp.int32)
    code_iota = jax.lax.broadcasted_iota(jnp.int32, distances.shape, 1)

    enc = (code_iota == idx2d).astype(jnp.float32)
    enc_ref[...] = enc
    idx_ref[...] = idx2d

    qst_ref[...] = jax.lax.dot_general(enc, w, (((1,), (0,)), ((), ())))

    acc_ref[...] += jnp.sum(dmin, keepdims=True)
    cnt_ref[...] += jnp.sum(enc, axis=0, keepdims=True)

    @pl.when(i == NUM_BLOCKS - 1)
    def _finalize():
        rowsq = jnp.sum(w * w, axis=1, keepdims=True)
        cnt = cnt_ref[...]
        corr = (jax.lax.dot_general(cnt, rowsq, (((1,), (0,)), ((), ())))
                - jnp.sum(cnt * colsq_ref[...], keepdims=True))
        m = (acc_ref[...] + corr) / jnp.float32(TOTAL_ROWS * LATENT_DIM)
        loss_ref[...] = m + jnp.float32(BETA) * m
        avg = cnt / jnp.float32(TOTAL_ROWS)
        ent = jnp.sum(avg * jnp.log(avg + 1e-10), keepdims=True)
        perp_ref[...] = jnp.exp(-ent)


@jax.jit
def kernel(x, W):
    xp = jnp.transpose(x, (0, 2, 3, 1))
    input_shape = xp.shape
    xf = xp.reshape(TOTAL_ROWS, LATENT_DIM)

    out_shapes = (
        jax.ShapeDtypeStruct((TOTAL_ROWS, LATENT_DIM), jnp.float32),
        jax.ShapeDtypeStruct((TOTAL_ROWS, CODEBOOK_SIZE), jnp.float32),
        jax.ShapeDtypeStruct((TOTAL_ROWS, 1), jnp.int32),
        jax.ShapeDtypeStruct((1, 1), jnp.float32),
        jax.ShapeDtypeStruct((1, 1), jnp.float32),
    )
    qst, enc, idx, loss, perp = pl.pallas_call(
        _vq_kernel,
        grid=(NUM_BLOCKS,),
        in_specs=[
            pl.BlockSpec((BLOCK_ROWS, LATENT_DIM), lambda i: (i, 0)),
            pl.BlockSpec((CODEBOOK_SIZE, LATENT_DIM), lambda i: (0, 0)),
        ],
        out_specs=(
            pl.BlockSpec((BLOCK_ROWS, LATENT_DIM), lambda i: (i, 0)),
            pl.BlockSpec((BLOCK_ROWS, CODEBOOK_SIZE), lambda i: (i, 0)),
            pl.BlockSpec((BLOCK_ROWS, 1), lambda i: (i, 0)),
            pl.BlockSpec((1, 1), lambda i: (0, 0)),
            pl.BlockSpec((1, 1), lambda i: (0, 0)),
        ),
        scratch_shapes=[
            pltpu.VMEM((1, 1), jnp.float32),
            pltpu.VMEM((1, CODEBOOK_SIZE), jnp.float32),
            pltpu.VMEM((1, CODEBOOK_SIZE), jnp.float32),
        ],
        out_shape=out_shapes,
    )(xf, W)

    q_out = jnp.transpose(qst.reshape(input_shape), (0, 3, 1, 2))
    return (loss[0, 0], q_out, perp[0, 0], enc, idx)


# speedup vs baseline: 2.6496x; 1.0600x over previous
import jax
import jax.numpy as jnp
from jax.experimental import pallas as pl
from jax.experimental.pallas import tpu as pltpu

LATENT_DIM = 1024
CODEBOOK_SIZE = 1024
BETA = 0.25
BLOCK_ROWS = 1024
TOTAL_ROWS = 8192
NUM_BLOCKS = TOTAL_ROWS // BLOCK_ROWS


def _vq_kernel(x_ref, w_ref, qst_ref, enc_ref, idx_ref, loss_ref, perp_ref,
               acc_ref, cnt_ref, colsq_ref):
    i = pl.program_id(0)
    xb = x_ref[...]
    w = w_ref[...]

    @pl.when(i == 0)
    def _init():
        colsq_ref[...] = jnp.sum(w * w, axis=0, keepdims=True)
        acc_ref[...] = jnp.zeros_like(acc_ref)
        cnt_ref[...] = jnp.zeros_like(cnt_ref)

    xsq = jnp.sum(xb * xb, axis=-1, keepdims=True)
    s = jax.lax.dot_general(xb, w, (((1,), (1,)), ((), ())))
    distances = (xsq + colsq_ref[...]) - 2.0 * s

    dmin = jnp.min(distances, axis=1, keepdims=True)
    code_iota = jax.lax.broadcasted_iota(jnp.int32, distances.shape, 1)
    idx2d = jnp.min(jnp.where(distances == dmin, code_iota, CODEBOOK_SIZE),
                    axis=1, keepdims=True)

    enc = (code_iota == idx2d).astype(jnp.float32)
    enc_ref[...] = enc
    idx_ref[...] = idx2d

    qst_ref[...] = jax.lax.dot_general(enc, w, (((1,), (0,)), ((), ())))

    acc_ref[...] += jnp.sum(dmin, keepdims=True)
    cnt_ref[...] += jnp.sum(enc, axis=0, keepdims=True)

    @pl.when(i == NUM_BLOCKS - 1)
    def _finalize():
        rowsq = jnp.sum(w * w, axis=1, keepdims=True)
        cnt = cnt_ref[...]
        corr = (jax.lax.dot_general(cnt, rowsq, (((1,), (0,)), ((), ())))
                - jnp.sum(cnt * colsq_ref[...], keepdims=True))
        m = (acc_ref[...] + corr) / jnp.float32(TOTAL_ROWS * LATENT_DIM)
        loss_ref[...] = m + jnp.float32(BETA) * m
        avg = cnt / jnp.float32(TOTAL_ROWS)
        ent = jnp.sum(avg * jnp.log(avg + 1e-10), keepdims=True)
        perp_ref[...] = jnp.exp(-ent)


@jax.jit
def kernel(x, W):
    xp = jnp.transpose(x, (0, 2, 3, 1))
    input_shape = xp.shape
    xf = xp.reshape(TOTAL_ROWS, LATENT_DIM)

    out_shapes = (
        jax.ShapeDtypeStruct((TOTAL_ROWS, LATENT_DIM), jnp.float32),
        jax.ShapeDtypeStruct((TOTAL_ROWS, CODEBOOK_SIZE), jnp.float32),
        jax.ShapeDtypeStruct((TOTAL_ROWS, 1), jnp.int32),
        jax.ShapeDtypeStruct((1, 1), jnp.float32),
        jax.ShapeDtypeStruct((1, 1), jnp.float32),
    )
    qst, enc, idx, loss, perp = pl.pallas_call(
        _vq_kernel,
        grid=(NUM_BLOCKS,),
        in_specs=[
            pl.BlockSpec((BLOCK_ROWS, LATENT_DIM), lambda i: (i, 0)),
            pl.BlockSpec((CODEBOOK_SIZE, LATENT_DIM), lambda i: (0, 0)),
        ],
        out_specs=(
            pl.BlockSpec((BLOCK_ROWS, LATENT_DIM), lambda i: (i, 0)),
            pl.BlockSpec((BLOCK_ROWS, CODEBOOK_SIZE), lambda i: (i, 0)),
            pl.BlockSpec((BLOCK_ROWS, 1), lambda i: (i, 0)),
            pl.BlockSpec((1, 1), lambda i: (0, 0)),
            pl.BlockSpec((1, 1), lambda i: (0, 0)),
        ),
        scratch_shapes=[
            pltpu.VMEM((1, 1), jnp.float32),
            pltpu.VMEM((1, CODEBOOK_SIZE), jnp.float32),
            pltpu.VMEM((1, CODEBOOK_SIZE), jnp.float32),
        ],
        out_shape=out_shapes,
    )(xf, W)

    q_out = jnp.transpose(qst.reshape(input_shape), (0, 3, 1, 2))
    return (loss[0, 0], q_out, perp[0, 0], enc, idx)
